# baseline (speedup 1.0000x reference)
import jax
import jax.numpy as jnp
from jax import lax
from jax.experimental import pallas as pl
from jax.experimental.pallas import tpu as pltpu
from jax.experimental.pallas import tpu_sc as plsc

N = 10000
NP = 10240
D = 128
E_TOT = 330000
NC, NS = 2, 16
NW = NC * NS
TE = 10752
EP = NW * TE
C = 112
KC = TE // C
VE = TE // 16
NB = 10
BN = NP // NB
RPT = NP // NS
RC = 80
NEG = -1e30

_f32 = jnp.float32
_i32 = jnp.int32

_sc_mesh = plsc.VectorSubcoreMesh(
    core_axis_name="c", subcore_axis_name="s", num_cores=NC, num_subcores=NS)
_sc_params = pltpu.CompilerParams(needs_layout_passes=False)


def _tc_embed_body(x_ref, w_ref, asv_ref, adv_ref, h_ref, as_ref, ad_ref):
    h = jnp.dot(x_ref[...], w_ref[...], preferred_element_type=_f32)
    h_ref[...] = h
    as_ref[...] = jnp.dot(h, asv_ref[...], preferred_element_type=_f32)
    ad_ref[...] = jnp.dot(h, adv_ref[...], preferred_element_type=_f32)


def _tc_prep2_body(p_ref, b_ref, w_ref, asv_ref, adv_ref,
                   h2_ref, as_ref, ad_ref):
    h1 = jnp.maximum(p_ref[0] + p_ref[1] + b_ref[...], 0.0)
    h2 = jnp.dot(h1, w_ref[...], preferred_element_type=_f32)
    h2_ref[...] = h2
    as_ref[...] = jnp.dot(h2, asv_ref[...], preferred_element_type=_f32)
    ad_ref[...] = jnp.dot(h2, adv_ref[...], preferred_element_type=_f32)


def _tc_combine_body(d_ref, m_ref, inv_ref, mout_ref):
    mt = m_ref[...]
    m = jnp.max(mt)
    scale = jnp.exp(mt[:, :1] - m)
    d = jnp.sum(d_ref[...] * scale, axis=0, keepdims=True)
    inv_ref[...] = 1.0 / (d + 1e-16)
    mout_ref[...] = jnp.full((1, 16), m, _f32)


def _tc_final_body(wt_ref, h2_ref, b_ref, o_ref):
    w = jnp.sum(wt_ref[...], axis=0, keepdims=True)
    o = jnp.dot(w, h2_ref[...], preferred_element_type=_f32) * (1.0 / N)
    o_ref[...] = o + b_ref[...]


def _embed(x, W, asv, adv):
    return pl.pallas_call(
        _tc_embed_body,
        grid=(NB,),
        in_specs=[pl.BlockSpec((BN, D), lambda i: (i, 0)),
                  pl.BlockSpec((D, D), lambda i: (0, 0)),
                  pl.BlockSpec((D, 1), lambda i: (0, 0)),
                  pl.BlockSpec((D, 1), lambda i: (0, 0))],
        out_specs=[pl.BlockSpec((BN, D), lambda i: (i, 0)),
                   pl.BlockSpec((BN, 1), lambda i: (i, 0)),
                   pl.BlockSpec((BN, 1), lambda i: (i, 0))],
        out_shape=[jax.ShapeDtypeStruct((NP, D), _f32),
                   jax.ShapeDtypeStruct((NP, 1), _f32),
                   jax.ShapeDtypeStruct((NP, 1), _f32)],
    )(x, W, asv, adv)


def _prep2(p, b1, W2, asv, adv):
    return pl.pallas_call(
        _tc_prep2_body,
        grid=(NB,),
        in_specs=[pl.BlockSpec((NC, BN, D), lambda i: (0, i, 0)),
                  pl.BlockSpec((1, D), lambda i: (0, 0)),
                  pl.BlockSpec((D, D), lambda i: (0, 0)),
                  pl.BlockSpec((D, 1), lambda i: (0, 0)),
                  pl.BlockSpec((D, 1), lambda i: (0, 0))],
        out_specs=[pl.BlockSpec((BN, D), lambda i: (i, 0)),
                   pl.BlockSpec((BN, 1), lambda i: (i, 0)),
                   pl.BlockSpec((BN, 1), lambda i: (i, 0))],
        out_shape=[jax.ShapeDtypeStruct((NP, D), _f32),
                   jax.ShapeDtypeStruct((NP, 1), _f32),
                   jax.ShapeDtypeStruct((NP, 1), _f32)],
    )(p, b1, W2, asv, adv)


def _combine(dt, mt):
    return pl.pallas_call(
        _tc_combine_body,
        out_shape=[jax.ShapeDtypeStruct((1, NP), _f32),
                   jax.ShapeDtypeStruct((1, 16), _f32)],
    )(dt, mt)


def _final(wt, h2, b2):
    return pl.pallas_call(
        _tc_final_body,
        out_shape=jax.ShapeDtypeStruct((1, D), _f32),
    )(wt, h2, b2)


def _sc_edge_body(src_hbm, dst_hbm, as_hbm, ad_hbm,
                  e_hbm, d_hbm, m_hbm,
                  src_v, dst_v, e_v, as_v, ad_v, d_v, m_v):
    cid = lax.axis_index("c")
    sid = lax.axis_index("s")
    wid = sid * NC + cid
    base = wid * TE
    pltpu.sync_copy(src_hbm.at[pl.ds(base, TE)], src_v)
    pltpu.sync_copy(dst_hbm.at[pl.ds(base, TE)], dst_v)
    pltpu.sync_copy(as_hbm, as_v)
    pltpu.sync_copy(ad_hbm, ad_v)

    zv = jnp.zeros((16,), _f32)

    def zero(i, _):
        for u in range(8):
            d_v[pl.ds(i * 128 + u * 16, 16)] = zv
        return 0
    lax.fori_loop(0, NP // 128, zero, 0)

    lanes = lax.iota(_i32, 16)

    def p1(i, mrun):
        for u in range(4):
            sl = pl.ds(i * 64 + u * 16, 16)
            a = plsc.load_gather(as_v, [src_v[sl]])
            b = plsc.load_gather(ad_v, [dst_v[sl]])
            e = a + b
            e = jnp.where(e >= 0.0, e, 0.2 * e)
            gid = base + i * 64 + u * 16 + lanes
            e = jnp.where(gid < E_TOT, e, NEG)
            e_v[sl] = e
            mrun = jnp.maximum(mrun, e)
        return mrun
    mrun = lax.fori_loop(0, VE // 4, p1, jnp.full((16,), NEG, _f32))
    mt = jnp.maximum(jnp.max(mrun), -1e20)

    def p2(i, _):
        for u in range(4):
            sl = pl.ds(i * 64 + u * 16, 16)
            ex = jnp.exp(e_v[sl] - mt)
            plsc.addupdate_scatter(d_v, [dst_v[sl]], ex)
        return 0
    lax.fori_loop(0, VE // 4, p2, 0)

    m_v[...] = jnp.full((16,), mt, _f32)
    pltpu.sync_copy(e_v, e_hbm.at[pl.ds(base, TE)])
    pltpu.sync_copy(d_v, d_hbm.at[wid])
    pltpu.sync_copy(m_v, m_hbm.at[wid])


def _sc_alpha_body(dst_hbm, e_hbm, inv_hbm, m_hbm,
                   a_hbm,
                   dst_v, e_v, a_v, inv_v, m_v):
    cid = lax.axis_index("c")
    sid = lax.axis_index("s")
    wid = sid * NC + cid
    base = wid * TE
    pltpu.sync_copy(dst_hbm.at[pl.ds(base, TE)], dst_v)
    pltpu.sync_copy(e_hbm.at[pl.ds(base, TE)], e_v)
    pltpu.sync_copy(inv_hbm, inv_v)
    pltpu.sync_copy(m_hbm, m_v)
    mvec = m_v[...]

    def p(i, _):
        for u in range(4):
            sl = pl.ds(i * 64 + u * 16, 16)
            iv = plsc.load_gather(inv_v, [dst_v[sl]])
            a_v[sl] = jnp.exp(e_v[sl] - mvec) * iv
        return 0
    lax.fori_loop(0, VE // 4, p, 0)

    pltpu.sync_copy(a_v, a_hbm.at[pl.ds(base, TE)])


def _sc_w_body(src_hbm, dst_hbm, e_hbm, inv_hbm, m_hbm,
               w_hbm,
               src_v, dst_v, e_v, inv_v, m_v, w_v):
    cid = lax.axis_index("c")
    sid = lax.axis_index("s")
    wid = sid * NC + cid
    base = wid * TE
    pltpu.sync_copy(src_hbm.at[pl.ds(base, TE)], src_v)
    pltpu.sync_copy(dst_hbm.at[pl.ds(base, TE)], dst_v)
    pltpu.sync_copy(e_hbm.at[pl.ds(base, TE)], e_v)
    pltpu.sync_copy(inv_hbm, inv_v)
    pltpu.sync_copy(m_hbm, m_v)
    mvec = m_v[...]

    zv = jnp.zeros((16,), _f32)

    def zero(i, _):
        for u in range(8):
            w_v[pl.ds(i * 128 + u * 16, 16)] = zv
        return 0
    lax.fori_loop(0, NP // 128, zero, 0)

    def p(i, _):
        for u in range(4):
            sl = pl.ds(i * 64 + u * 16, 16)
            iv = plsc.load_gather(inv_v, [dst_v[sl]])
            a = jnp.exp(e_v[sl] - mvec) * iv
            plsc.addupdate_scatter(w_v, [src_v[sl]], a)
        return 0
    lax.fori_loop(0, VE // 4, p, 0)

    pltpu.sync_copy(w_v, w_hbm.at[wid])


def _sc_heavy_body(src_hbm, dst_hbm, a_hbm, h_hbm,
                   out_hbm,
                   src_cs, dst_cs, a_cs, rows, acc,
                   gsem0, gsem1, gsem2, ssem0, ssem1, ssem2,
                   isem0, isem1, isem2, isem3, isem4, isem5):
    gsem = [gsem0, gsem1, gsem2]
    ssem = [ssem0, ssem1, ssem2]
    isem = [isem0, isem1, isem2, isem3, isem4, isem5]
    cid = lax.axis_index("c")
    sid = lax.axis_index("s")
    wid = sid * NC + cid
    base = wid * TE

    def zr(k, _):
        for f in range(D // 16):
            rows[0, k, pl.ds(f * 16, 16)] = jnp.zeros((16,), _f32)
        return 0
    lax.fori_loop(0, C, zr, 0)
    for j in range(RPT // RC):
        pltpu.sync_copy(rows.at[0, pl.ds(0, RC)],
                        acc.at[pl.ds(sid * RPT + j * RC, RC)])
    plsc.subcore_barrier()

    def idx_issue(c, qb):
        eb = base + c * C
        pltpu.async_copy(src_hbm.at[pl.ds(eb, C)], src_cs.at[qb], isem[qb])
        pltpu.async_copy(dst_hbm.at[pl.ds(eb, C)], dst_cs.at[qb], isem[qb])
        pltpu.async_copy(a_hbm.at[pl.ds(eb, C)], a_cs.at[qb], isem[qb])

    def idx_wait(c, qb):
        eb = base + c * C
        pltpu.make_async_copy(src_hbm.at[pl.ds(eb, C)], src_cs.at[qb],
                              isem[qb]).wait()
        pltpu.make_async_copy(dst_hbm.at[pl.ds(eb, C)], dst_cs.at[qb],
                              isem[qb]).wait()
        pltpu.make_async_copy(a_hbm.at[pl.ds(eb, C)], a_cs.at[qb],
                              isem[qb]).wait()

    def gather_issue(qb, rb):
        pltpu.async_copy(h_hbm.at[src_cs.at[qb]], rows.at[rb], gsem[rb])

    def gather_wait(qb, rb):
        pltpu.make_async_copy(h_hbm.at[src_cs.at[qb]], rows.at[rb],
                              gsem[rb]).wait()

    def scat_issue(qb, rb):
        pltpu.async_copy(rows.at[rb], acc.at[dst_cs.at[qb]], ssem[rb],
                        add=True)

    def scat_wait(qb, rb):
        pltpu.make_async_copy(rows.at[rb], acc.at[dst_cs.at[qb]],
                              ssem[rb]).wait()

    def scale(qb, rb):
        def g_body(g, _):
            av16 = a_cs[qb, pl.ds(g * 16, 16)]
            for k2 in range(16):
                av = jnp.full((16,), av16[k2], _f32)
                def row_mul(gk):
                    for f in range(D // 16):
                        sl = pl.ds(f * 16, 16)
                        rows[rb, gk, sl] = rows[rb, gk, sl] * av
                row_mul(g * 16 + k2)
            return 0
        lax.fori_loop(0, C // 16, g_body, 0)

    def step(c, qb, rb, do_swait, do_gissue, do_iissue, do_iwait):
        if do_iwait:
            idx_wait(c + 1, (qb + 1) % 6)
        if do_swait:
            scat_wait((qb + 4) % 6, (rb + 1) % 3)
        if do_gissue:
            gather_issue((qb + 1) % 6, (rb + 1) % 3)
        if do_iissue:
            idx_issue(c + 3, (qb + 3) % 6)
        gather_wait(qb, rb)
        scale(qb, rb)
        scat_issue(qb, rb)

    idx_issue(0, 0)
    idx_issue(1, 1)
    idx_issue(2, 2)
    idx_wait(0, 0)
    gather_issue(0, 0)

    for c in range(6):
        step(c, c % 6, c % 3, do_swait=(c >= 2), do_gissue=True,
             do_iissue=True, do_iwait=True)

    def body(k, _):
        c0 = k * 6
        for u in range(6):
            step(c0 + u, u, u % 3, do_swait=True, do_gissue=True,
                 do_iissue=True, do_iwait=True)
        return 0
    lax.fori_loop(1, KC // 6 - 1, body, 0)

    for c in range(KC - 6, KC):
        step(c, c % 6, c % 3, do_swait=True,
             do_gissue=(c + 1 < KC),
             do_iissue=(c + 3 < KC),
             do_iwait=(c + 1 < KC))

    scat_wait((KC - 2) % 6, (KC - 2) % 3)
    scat_wait((KC - 1) % 6, (KC - 1) % 3)
    plsc.subcore_barrier()

    for j in range(RPT // RC):
        r = sid * RPT + j * RC
        pltpu.sync_copy(acc.at[pl.ds(r, RC)], rows.at[0, pl.ds(0, RC)])
        pltpu.sync_copy(rows.at[0, pl.ds(0, RC)], out_hbm.at[cid, pl.ds(r, RC)])


_edge_pass = pl.kernel(
    _sc_edge_body,
    out_type=(jax.ShapeDtypeStruct((EP,), _f32),
              jax.ShapeDtypeStruct((NW, NP), _f32),
              jax.ShapeDtypeStruct((NW, 16), _f32)),
    mesh=_sc_mesh,
    compiler_params=_sc_params,
    scratch_types=[pltpu.VMEM((TE,), _i32),
                   pltpu.VMEM((TE,), _i32),
                   pltpu.VMEM((TE,), _f32),
                   pltpu.VMEM((NP,), _f32),
                   pltpu.VMEM((NP,), _f32),
                   pltpu.VMEM((NP,), _f32),
                   pltpu.VMEM((16,), _f32)],
)

_alpha_pass = pl.kernel(
    _sc_alpha_body,
    out_type=jax.ShapeDtypeStruct((EP,), _f32),
    mesh=_sc_mesh,
    compiler_params=_sc_params,
    scratch_types=[pltpu.VMEM((TE,), _i32),
                   pltpu.VMEM((TE,), _f32),
                   pltpu.VMEM((TE,), _f32),
                   pltpu.VMEM((NP,), _f32),
                   pltpu.VMEM((16,), _f32)],
)

_wpass = pl.kernel(
    _sc_w_body,
    out_type=jax.ShapeDtypeStruct((NW, NP), _f32),
    mesh=_sc_mesh,
    compiler_params=_sc_params,
    scratch_types=[pltpu.VMEM((TE,), _i32),
                   pltpu.VMEM((TE,), _i32),
                   pltpu.VMEM((TE,), _f32),
                   pltpu.VMEM((NP,), _f32),
                   pltpu.VMEM((16,), _f32),
                   pltpu.VMEM((NP,), _f32)],
)

_heavy = pl.kernel(
    _sc_heavy_body,
    out_type=jax.ShapeDtypeStruct((NC, NP, D), _f32),
    mesh=_sc_mesh,
    compiler_params=_sc_params,
    scratch_types=[pltpu.VMEM((6, C), _i32),
                   pltpu.VMEM((6, C), _i32),
                   pltpu.VMEM((6, C), _f32),
                   pltpu.VMEM((3, C, D), _f32),
                   pltpu.VMEM_SHARED((NP, D), _f32)]
                  + [pltpu.SemaphoreType.DMA] * 12,
)


def kernel(x, edge_index, W1, att_src1, att_dst1, bias1,
           W2, att_src2, att_dst2, bias2):
    ei = edge_index.astype(_i32)
    loop = jnp.arange(N, dtype=_i32)
    pad = jnp.arange(EP - E_TOT, dtype=_i32) % N
    src = jnp.concatenate([ei[0], loop, pad])
    dst = jnp.concatenate([ei[1], loop, pad])
    xp = jnp.pad(x, ((0, NP - N), (0, 0)))

    h1, as1, ad1 = _embed(xp, W1, att_src1.reshape(D, 1), att_dst1.reshape(D, 1))
    e1, dt1, mt1 = _edge_pass(src, dst, as1.reshape(NP), ad1.reshape(NP))
    inv1, m1 = _combine(dt1, mt1)
    a1 = _alpha_pass(dst, e1, inv1.reshape(NP), m1.reshape(16))
    outp = _heavy(src, dst, a1, h1)

    h2, as2, ad2 = _prep2(outp, bias1.reshape(1, D), W2,
                          att_src2.reshape(D, 1), att_dst2.reshape(D, 1))
    e2, dt2, mt2 = _edge_pass(src, dst, as2.reshape(NP), ad2.reshape(NP))
    inv2, m2 = _combine(dt2, mt2)
    wt = _wpass(src, dst, e2, inv2.reshape(NP), m2.reshape(16))
    out = _final(wt, h2, bias2.reshape(1, D))
    return out.reshape(D)


# speedup vs baseline: 1.0135x; 1.0135x over previous
import jax
import jax.numpy as jnp
from jax import lax
from jax.experimental import pallas as pl
from jax.experimental.pallas import tpu as pltpu
from jax.experimental.pallas import tpu_sc as plsc

N = 10000
NP = 10240
D = 128
E_TOT = 330000
NC, NS = 2, 16
NW = NC * NS
TE = 10368
EP = NW * TE
C = 96
KC = TE // C
VE = TE // 16
NB = 10
BN = NP // NB
RPT = NP // NS
RC = 80
NEG = -1e30

_f32 = jnp.float32
_i32 = jnp.int32

_sc_mesh = plsc.VectorSubcoreMesh(
    core_axis_name="c", subcore_axis_name="s", num_cores=NC, num_subcores=NS)
_sc_params = pltpu.CompilerParams(needs_layout_passes=False)


def _tc_embed_body(x_ref, w_ref, asv_ref, adv_ref, h_ref, as_ref, ad_ref):
    h = jnp.dot(x_ref[...], w_ref[...], preferred_element_type=_f32)
    h_ref[...] = h
    as_ref[...] = jnp.dot(h, asv_ref[...], preferred_element_type=_f32)
    ad_ref[...] = jnp.dot(h, adv_ref[...], preferred_element_type=_f32)


def _tc_prep2_body(p_ref, b_ref, w_ref, asv_ref, adv_ref,
                   h2_ref, as_ref, ad_ref):
    h1 = jnp.maximum(p_ref[0] + p_ref[1] + b_ref[...], 0.0)
    h2 = jnp.dot(h1, w_ref[...], preferred_element_type=_f32)
    h2_ref[...] = h2
    as_ref[...] = jnp.dot(h2, asv_ref[...], preferred_element_type=_f32)
    ad_ref[...] = jnp.dot(h2, adv_ref[...], preferred_element_type=_f32)


def _tc_combine_body(d_ref, m_ref, inv_ref, mout_ref):
    mt = m_ref[...]
    m = jnp.max(mt)
    scale = jnp.exp(mt[:, :1] - m)
    d = jnp.sum(d_ref[...] * scale, axis=0, keepdims=True)
    inv_ref[...] = 1.0 / (d + 1e-16)
    mout_ref[...] = jnp.full((1, 16), m, _f32)


def _tc_final_body(wt_ref, h2_ref, b_ref, o_ref):
    w = jnp.sum(wt_ref[...], axis=0, keepdims=True)
    o = jnp.dot(w, h2_ref[...], preferred_element_type=_f32) * (1.0 / N)
    o_ref[...] = o + b_ref[...]


def _embed(x, W, asv, adv):
    return pl.pallas_call(
        _tc_embed_body,
        grid=(NB,),
        in_specs=[pl.BlockSpec((BN, D), lambda i: (i, 0)),
                  pl.BlockSpec((D, D), lambda i: (0, 0)),
                  pl.BlockSpec((D, 1), lambda i: (0, 0)),
                  pl.BlockSpec((D, 1), lambda i: (0, 0))],
        out_specs=[pl.BlockSpec((BN, D), lambda i: (i, 0)),
                   pl.BlockSpec((BN, 1), lambda i: (i, 0)),
                   pl.BlockSpec((BN, 1), lambda i: (i, 0))],
        out_shape=[jax.ShapeDtypeStruct((NP, D), _f32),
                   jax.ShapeDtypeStruct((NP, 1), _f32),
                   jax.ShapeDtypeStruct((NP, 1), _f32)],
    )(x, W, asv, adv)


def _prep2(p, b1, W2, asv, adv):
    return pl.pallas_call(
        _tc_prep2_body,
        grid=(NB,),
        in_specs=[pl.BlockSpec((NC, BN, D), lambda i: (0, i, 0)),
                  pl.BlockSpec((1, D), lambda i: (0, 0)),
                  pl.BlockSpec((D, D), lambda i: (0, 0)),
                  pl.BlockSpec((D, 1), lambda i: (0, 0)),
                  pl.BlockSpec((D, 1), lambda i: (0, 0))],
        out_specs=[pl.BlockSpec((BN, D), lambda i: (i, 0)),
                   pl.BlockSpec((BN, 1), lambda i: (i, 0)),
                   pl.BlockSpec((BN, 1), lambda i: (i, 0))],
        out_shape=[jax.ShapeDtypeStruct((NP, D), _f32),
                   jax.ShapeDtypeStruct((NP, 1), _f32),
                   jax.ShapeDtypeStruct((NP, 1), _f32)],
    )(p, b1, W2, asv, adv)


def _combine(dt, mt):
    return pl.pallas_call(
        _tc_combine_body,
        out_shape=[jax.ShapeDtypeStruct((1, NP), _f32),
                   jax.ShapeDtypeStruct((1, 16), _f32)],
    )(dt, mt)


def _final(wt, h2, b2):
    return pl.pallas_call(
        _tc_final_body,
        out_shape=jax.ShapeDtypeStruct((1, D), _f32),
    )(wt, h2, b2)


def _sc_edge_body(src_hbm, dst_hbm, as_hbm, ad_hbm,
                  e_hbm, d_hbm, m_hbm,
                  src_v, dst_v, e_v, as_v, ad_v, d_v, m_v):
    cid = lax.axis_index("c")
    sid = lax.axis_index("s")
    wid = sid * NC + cid
    base = wid * TE
    pltpu.sync_copy(src_hbm.at[pl.ds(base, TE)], src_v)
    pltpu.sync_copy(dst_hbm.at[pl.ds(base, TE)], dst_v)
    pltpu.sync_copy(as_hbm, as_v)
    pltpu.sync_copy(ad_hbm, ad_v)

    zv = jnp.zeros((16,), _f32)

    def zero(i, _):
        for u in range(8):
            d_v[pl.ds(i * 128 + u * 16, 16)] = zv
        return 0
    lax.fori_loop(0, NP // 128, zero, 0)

    lanes = lax.iota(_i32, 16)

    def p1(i, mrun):
        for u in range(4):
            sl = pl.ds(i * 64 + u * 16, 16)
            a = plsc.load_gather(as_v, [src_v[sl]])
            b = plsc.load_gather(ad_v, [dst_v[sl]])
            e = a + b
            e = jnp.where(e >= 0.0, e, 0.2 * e)
            gid = base + i * 64 + u * 16 + lanes
            e = jnp.where(gid < E_TOT, e, NEG)
            e_v[sl] = e
            mrun = jnp.maximum(mrun, e)
        return mrun
    mrun = lax.fori_loop(0, VE // 4, p1, jnp.full((16,), NEG, _f32))
    mt = jnp.maximum(jnp.max(mrun), -1e20)

    def p2(i, _):
        for u in range(4):
            sl = pl.ds(i * 64 + u * 16, 16)
            ex = jnp.exp(e_v[sl] - mt)
            plsc.addupdate_scatter(d_v, [dst_v[sl]], ex)
        return 0
    lax.fori_loop(0, VE // 4, p2, 0)

    m_v[...] = jnp.full((16,), mt, _f32)
    pltpu.sync_copy(e_v, e_hbm.at[pl.ds(base, TE)])
    pltpu.sync_copy(d_v, d_hbm.at[wid])
    pltpu.sync_copy(m_v, m_hbm.at[wid])


def _sc_alpha_body(dst_hbm, e_hbm, inv_hbm, m_hbm,
                   a_hbm,
                   dst_v, e_v, a_v, inv_v, m_v):
    cid = lax.axis_index("c")
    sid = lax.axis_index("s")
    wid = sid * NC + cid
    base = wid * TE
    pltpu.sync_copy(dst_hbm.at[pl.ds(base, TE)], dst_v)
    pltpu.sync_copy(e_hbm.at[pl.ds(base, TE)], e_v)
    pltpu.sync_copy(inv_hbm, inv_v)
    pltpu.sync_copy(m_hbm, m_v)
    mvec = m_v[...]

    def p(i, _):
        for u in range(4):
            sl = pl.ds(i * 64 + u * 16, 16)
            iv = plsc.load_gather(inv_v, [dst_v[sl]])
            a_v[sl] = jnp.exp(e_v[sl] - mvec) * iv
        return 0
    lax.fori_loop(0, VE // 4, p, 0)

    pltpu.sync_copy(a_v, a_hbm.at[pl.ds(base, TE)])


def _sc_w_body(src_hbm, dst_hbm, e_hbm, inv_hbm, m_hbm,
               w_hbm,
               src_v, dst_v, e_v, inv_v, m_v, w_v):
    cid = lax.axis_index("c")
    sid = lax.axis_index("s")
    wid = sid * NC + cid
    base = wid * TE
    pltpu.sync_copy(src_hbm.at[pl.ds(base, TE)], src_v)
    pltpu.sync_copy(dst_hbm.at[pl.ds(base, TE)], dst_v)
    pltpu.sync_copy(e_hbm.at[pl.ds(base, TE)], e_v)
    pltpu.sync_copy(inv_hbm, inv_v)
    pltpu.sync_copy(m_hbm, m_v)
    mvec = m_v[...]

    zv = jnp.zeros((16,), _f32)

    def zero(i, _):
        for u in range(8):
            w_v[pl.ds(i * 128 + u * 16, 16)] = zv
        return 0
    lax.fori_loop(0, NP // 128, zero, 0)

    def p(i, _):
        for u in range(4):
            sl = pl.ds(i * 64 + u * 16, 16)
            iv = plsc.load_gather(inv_v, [dst_v[sl]])
            a = jnp.exp(e_v[sl] - mvec) * iv
            plsc.addupdate_scatter(w_v, [src_v[sl]], a)
        return 0
    lax.fori_loop(0, VE // 4, p, 0)

    pltpu.sync_copy(w_v, w_hbm.at[wid])


def _sc_heavy_body(src_hbm, dst_hbm, a_hbm, h_hbm,
                   out_hbm,
                   src_cs, dst_cs, a_cs, rows, acc,
                   gsem0, gsem1, gsem2, ssem0, ssem1, ssem2,
                   isem0, isem1, isem2, isem3, isem4, isem5):
    gsem = [gsem0, gsem1, gsem2]
    ssem = [ssem0, ssem1, ssem2]
    isem = [isem0, isem1, isem2, isem3, isem4, isem5]
    cid = lax.axis_index("c")
    sid = lax.axis_index("s")
    wid = sid * NC + cid
    base = wid * TE

    def zr(k, _):
        for f in range(D // 16):
            rows[0, k, pl.ds(f * 16, 16)] = jnp.zeros((16,), _f32)
        return 0
    lax.fori_loop(0, C, zr, 0)
    for j in range(RPT // RC):
        pltpu.sync_copy(rows.at[0, pl.ds(0, RC)],
                        acc.at[pl.ds(sid * RPT + j * RC, RC)])
    plsc.subcore_barrier()

    def idx_issue(c, qb):
        eb = base + c * C
        pltpu.async_copy(src_hbm.at[pl.ds(eb, C)], src_cs.at[qb], isem[qb])
        pltpu.async_copy(dst_hbm.at[pl.ds(eb, C)], dst_cs.at[qb], isem[qb])
        pltpu.async_copy(a_hbm.at[pl.ds(eb, C)], a_cs.at[qb], isem[qb])

    def idx_wait(c, qb):
        eb = base + c * C
        pltpu.make_async_copy(src_hbm.at[pl.ds(eb, C)], src_cs.at[qb],
                              isem[qb]).wait()
        pltpu.make_async_copy(dst_hbm.at[pl.ds(eb, C)], dst_cs.at[qb],
                              isem[qb]).wait()
        pltpu.make_async_copy(a_hbm.at[pl.ds(eb, C)], a_cs.at[qb],
                              isem[qb]).wait()

    def gather_issue(qb, rb):
        pltpu.async_copy(h_hbm.at[src_cs.at[qb]], rows.at[rb], gsem[rb])

    def gather_wait(qb, rb):
        pltpu.make_async_copy(h_hbm.at[src_cs.at[qb]], rows.at[rb],
                              gsem[rb]).wait()

    def scat_issue(qb, rb):
        pltpu.async_copy(rows.at[rb], acc.at[dst_cs.at[qb]], ssem[rb],
                        add=True)

    def scat_wait(qb, rb):
        pltpu.make_async_copy(rows.at[rb], acc.at[dst_cs.at[qb]],
                              ssem[rb]).wait()

    def scale(qb, rb):
        def g_body(g, _):
            av16 = a_cs[qb, pl.ds(g * 16, 16)]
            for k2 in range(16):
                av = jnp.full((16,), av16[k2], _f32)
                def row_mul(gk):
                    for f in range(D // 16):
                        sl = pl.ds(f * 16, 16)
                        rows[rb, gk, sl] = rows[rb, gk, sl] * av
                row_mul(g * 16 + k2)
            return 0
        lax.fori_loop(0, C // 16, g_body, 0)

    def step(c, qb, rb, do_swait, do_gissue, do_iissue, do_iwait):
        if do_iwait:
            idx_wait(c + 1, (qb + 1) % 6)
        if do_swait:
            scat_wait((qb + 4) % 6, (rb + 1) % 3)
        if do_gissue:
            gather_issue((qb + 1) % 6, (rb + 1) % 3)
        if do_iissue:
            idx_issue(c + 3, (qb + 3) % 6)
        gather_wait(qb, rb)
        scale(qb, rb)
        scat_issue(qb, rb)

    idx_issue(0, 0)
    idx_issue(1, 1)
    idx_issue(2, 2)
    idx_wait(0, 0)
    gather_issue(0, 0)

    for c in range(6):
        step(c, c % 6, c % 3, do_swait=(c >= 2), do_gissue=True,
             do_iissue=True, do_iwait=True)

    def body(k, _):
        c0 = k * 6
        for u in range(6):
            step(c0 + u, u, u % 3, do_swait=True, do_gissue=True,
                 do_iissue=True, do_iwait=True)
        return 0
    lax.fori_loop(1, KC // 6 - 1, body, 0)

    for c in range(KC - 6, KC):
        step(c, c % 6, c % 3, do_swait=True,
             do_gissue=(c + 1 < KC),
             do_iissue=(c + 3 < KC),
             do_iwait=(c + 1 < KC))

    scat_wait((KC - 2) % 6, (KC - 2) % 3)
    scat_wait((KC - 1) % 6, (KC - 1) % 3)
    plsc.subcore_barrier()

    for j in range(RPT // RC):
        r = sid * RPT + j * RC
        pltpu.sync_copy(acc.at[pl.ds(r, RC)], rows.at[0, pl.ds(0, RC)])
        pltpu.sync_copy(rows.at[0, pl.ds(0, RC)], out_hbm.at[cid, pl.ds(r, RC)])


_edge_pass = pl.kernel(
    _sc_edge_body,
    out_type=(jax.ShapeDtypeStruct((EP,), _f32),
              jax.ShapeDtypeStruct((NW, NP), _f32),
              jax.ShapeDtypeStruct((NW, 16), _f32)),
    mesh=_sc_mesh,
    compiler_params=_sc_params,
    scratch_types=[pltpu.VMEM((TE,), _i32),
                   pltpu.VMEM((TE,), _i32),
                   pltpu.VMEM((TE,), _f32),
                   pltpu.VMEM((NP,), _f32),
                   pltpu.VMEM((NP,), _f32),
                   pltpu.VMEM((NP,), _f32),
                   pltpu.VMEM((16,), _f32)],
)

_alpha_pass = pl.kernel(
    _sc_alpha_body,
    out_type=jax.ShapeDtypeStruct((EP,), _f32),
    mesh=_sc_mesh,
    compiler_params=_sc_params,
    scratch_types=[pltpu.VMEM((TE,), _i32),
                   pltpu.VMEM((TE,), _f32),
                   pltpu.VMEM((TE,), _f32),
                   pltpu.VMEM((NP,), _f32),
                   pltpu.VMEM((16,), _f32)],
)

_wpass = pl.kernel(
    _sc_w_body,
    out_type=jax.ShapeDtypeStruct((NW, NP), _f32),
    mesh=_sc_mesh,
    compiler_params=_sc_params,
    scratch_types=[pltpu.VMEM((TE,), _i32),
                   pltpu.VMEM((TE,), _i32),
                   pltpu.VMEM((TE,), _f32),
                   pltpu.VMEM((NP,), _f32),
                   pltpu.VMEM((16,), _f32),
                   pltpu.VMEM((NP,), _f32)],
)

_heavy = pl.kernel(
    _sc_heavy_body,
    out_type=jax.ShapeDtypeStruct((NC, NP, D), _f32),
    mesh=_sc_mesh,
    compiler_params=_sc_params,
    scratch_types=[pltpu.VMEM((6, C), _i32),
                   pltpu.VMEM((6, C), _i32),
                   pltpu.VMEM((6, C), _f32),
                   pltpu.VMEM((3, C, D), _f32),
                   pltpu.VMEM_SHARED((NP, D), _f32)]
                  + [pltpu.SemaphoreType.DMA] * 12,
)


def kernel(x, edge_index, W1, att_src1, att_dst1, bias1,
           W2, att_src2, att_dst2, bias2):
    ei = edge_index.astype(_i32)
    loop = jnp.arange(N, dtype=_i32)
    pad = jnp.arange(EP - E_TOT, dtype=_i32) % N
    src = jnp.concatenate([ei[0], loop, pad])
    dst = jnp.concatenate([ei[1], loop, pad])
    xp = jnp.pad(x, ((0, NP - N), (0, 0)))

    h1, as1, ad1 = _embed(xp, W1, att_src1.reshape(D, 1), att_dst1.reshape(D, 1))
    e1, dt1, mt1 = _edge_pass(src, dst, as1.reshape(NP), ad1.reshape(NP))
    inv1, m1 = _combine(dt1, mt1)
    a1 = _alpha_pass(dst, e1, inv1.reshape(NP), m1.reshape(16))
    outp = _heavy(src, dst, a1, h1)

    h2, as2, ad2 = _prep2(outp, bias1.reshape(1, D), W2,
                          att_src2.reshape(D, 1), att_dst2.reshape(D, 1))
    e2, dt2, mt2 = _edge_pass(src, dst, as2.reshape(NP), ad2.reshape(NP))
    inv2, m2 = _combine(dt2, mt2)
    wt = _wpass(src, dst, e2, inv2.reshape(NP), m2.reshape(16))
    out = _final(wt, h2, bias2.reshape(1, D))
    return out.reshape(D)


# speedup vs baseline: 1.0147x; 1.0011x over previous
import jax
import jax.numpy as jnp
from jax import lax
from jax.experimental import pallas as pl
from jax.experimental.pallas import tpu as pltpu
from jax.experimental.pallas import tpu_sc as plsc

N = 10000
NP = 10240
D = 128
E_TOT = 330000
NC, NS = 2, 16
NW = NC * NS
TE = 10368
EP = NW * TE
C = 64
KC = TE // C
VE = TE // 16
NB = 10
BN = NP // NB
RPT = NP // NS
RC = 64
NEG = -1e30

_f32 = jnp.float32
_i32 = jnp.int32

_sc_mesh = plsc.VectorSubcoreMesh(
    core_axis_name="c", subcore_axis_name="s", num_cores=NC, num_subcores=NS)
_sc_params = pltpu.CompilerParams(needs_layout_passes=False)


def _tc_embed_body(x_ref, w_ref, asv_ref, adv_ref, h_ref, as_ref, ad_ref):
    h = jnp.dot(x_ref[...], w_ref[...], preferred_element_type=_f32)
    h_ref[...] = h
    as_ref[...] = jnp.dot(h, asv_ref[...], preferred_element_type=_f32)
    ad_ref[...] = jnp.dot(h, adv_ref[...], preferred_element_type=_f32)


def _tc_prep2_body(p_ref, b_ref, w_ref, asv_ref, adv_ref,
                   h2_ref, as_ref, ad_ref):
    h1 = jnp.maximum(p_ref[0] + p_ref[1] + b_ref[...], 0.0)
    h2 = jnp.dot(h1, w_ref[...], preferred_element_type=_f32)
    h2_ref[...] = h2
    as_ref[...] = jnp.dot(h2, asv_ref[...], preferred_element_type=_f32)
    ad_ref[...] = jnp.dot(h2, adv_ref[...], preferred_element_type=_f32)


def _tc_combine_body(d_ref, m_ref, inv_ref, mout_ref):
    mt = m_ref[...]
    m = jnp.max(mt)
    scale = jnp.exp(mt[:, :1] - m)
    d = jnp.sum(d_ref[...] * scale, axis=0, keepdims=True)
    inv_ref[...] = 1.0 / (d + 1e-16)
    mout_ref[...] = jnp.full((1, 16), m, _f32)


def _tc_final_body(wt_ref, h2_ref, b_ref, o_ref):
    w = jnp.sum(wt_ref[...], axis=0, keepdims=True)
    o = jnp.dot(w, h2_ref[...], preferred_element_type=_f32) * (1.0 / N)
    o_ref[...] = o + b_ref[...]


def _embed(x, W, asv, adv):
    return pl.pallas_call(
        _tc_embed_body,
        grid=(NB,),
        in_specs=[pl.BlockSpec((BN, D), lambda i: (i, 0)),
                  pl.BlockSpec((D, D), lambda i: (0, 0)),
                  pl.BlockSpec((D, 1), lambda i: (0, 0)),
                  pl.BlockSpec((D, 1), lambda i: (0, 0))],
        out_specs=[pl.BlockSpec((BN, D), lambda i: (i, 0)),
                   pl.BlockSpec((BN, 1), lambda i: (i, 0)),
                   pl.BlockSpec((BN, 1), lambda i: (i, 0))],
        out_shape=[jax.ShapeDtypeStruct((NP, D), _f32),
                   jax.ShapeDtypeStruct((NP, 1), _f32),
                   jax.ShapeDtypeStruct((NP, 1), _f32)],
    )(x, W, asv, adv)


def _prep2(p, b1, W2, asv, adv):
    return pl.pallas_call(
        _tc_prep2_body,
        grid=(NB,),
        in_specs=[pl.BlockSpec((NC, BN, D), lambda i: (0, i, 0)),
                  pl.BlockSpec((1, D), lambda i: (0, 0)),
                  pl.BlockSpec((D, D), lambda i: (0, 0)),
                  pl.BlockSpec((D, 1), lambda i: (0, 0)),
                  pl.BlockSpec((D, 1), lambda i: (0, 0))],
        out_specs=[pl.BlockSpec((BN, D), lambda i: (i, 0)),
                   pl.BlockSpec((BN, 1), lambda i: (i, 0)),
                   pl.BlockSpec((BN, 1), lambda i: (i, 0))],
        out_shape=[jax.ShapeDtypeStruct((NP, D), _f32),
                   jax.ShapeDtypeStruct((NP, 1), _f32),
                   jax.ShapeDtypeStruct((NP, 1), _f32)],
    )(p, b1, W2, asv, adv)


def _combine(dt, mt):
    return pl.pallas_call(
        _tc_combine_body,
        out_shape=[jax.ShapeDtypeStruct((1, NP), _f32),
                   jax.ShapeDtypeStruct((1, 16), _f32)],
    )(dt, mt)


def _final(wt, h2, b2):
    return pl.pallas_call(
        _tc_final_body,
        out_shape=jax.ShapeDtypeStruct((1, D), _f32),
    )(wt, h2, b2)


def _sc_edge_body(src_hbm, dst_hbm, as_hbm, ad_hbm,
                  e_hbm, d_hbm, m_hbm,
                  src_v, dst_v, e_v, as_v, ad_v, d_v, m_v):
    cid = lax.axis_index("c")
    sid = lax.axis_index("s")
    wid = sid * NC + cid
    base = wid * TE
    pltpu.sync_copy(src_hbm.at[pl.ds(base, TE)], src_v)
    pltpu.sync_copy(dst_hbm.at[pl.ds(base, TE)], dst_v)
    pltpu.sync_copy(as_hbm, as_v)
    pltpu.sync_copy(ad_hbm, ad_v)

    zv = jnp.zeros((16,), _f32)

    def zero(i, _):
        for u in range(8):
            d_v[pl.ds(i * 128 + u * 16, 16)] = zv
        return 0
    lax.fori_loop(0, NP // 128, zero, 0)

    lanes = lax.iota(_i32, 16)

    def p1(i, mrun):
        for u in range(4):
            sl = pl.ds(i * 64 + u * 16, 16)
            a = plsc.load_gather(as_v, [src_v[sl]])
            b = plsc.load_gather(ad_v, [dst_v[sl]])
            e = a + b
            e = jnp.where(e >= 0.0, e, 0.2 * e)
            gid = base + i * 64 + u * 16 + lanes
            e = jnp.where(gid < E_TOT, e, NEG)
            e_v[sl] = e
            mrun = jnp.maximum(mrun, e)
        return mrun
    mrun = lax.fori_loop(0, VE // 4, p1, jnp.full((16,), NEG, _f32))
    mt = jnp.maximum(jnp.max(mrun), -1e20)

    def p2(i, _):
        for u in range(4):
            sl = pl.ds(i * 64 + u * 16, 16)
            ex = jnp.exp(e_v[sl] - mt)
            plsc.addupdate_scatter(d_v, [dst_v[sl]], ex)
        return 0
    lax.fori_loop(0, VE // 4, p2, 0)

    m_v[...] = jnp.full((16,), mt, _f32)
    pltpu.sync_copy(e_v, e_hbm.at[pl.ds(base, TE)])
    pltpu.sync_copy(d_v, d_hbm.at[wid])
    pltpu.sync_copy(m_v, m_hbm.at[wid])


def _sc_w_body(src_hbm, dst_hbm, e_hbm, inv_hbm, m_hbm,
               w_hbm,
               src_v, dst_v, e_v, inv_v, m_v, w_v):
    cid = lax.axis_index("c")
    sid = lax.axis_index("s")
    wid = sid * NC + cid
    base = wid * TE
    pltpu.sync_copy(src_hbm.at[pl.ds(base, TE)], src_v)
    pltpu.sync_copy(dst_hbm.at[pl.ds(base, TE)], dst_v)
    pltpu.sync_copy(e_hbm.at[pl.ds(base, TE)], e_v)
    pltpu.sync_copy(inv_hbm, inv_v)
    pltpu.sync_copy(m_hbm, m_v)
    mvec = m_v[...]

    zv = jnp.zeros((16,), _f32)

    def zero(i, _):
        for u in range(8):
            w_v[pl.ds(i * 128 + u * 16, 16)] = zv
        return 0
    lax.fori_loop(0, NP // 128, zero, 0)

    def p(i, _):
        for u in range(4):
            sl = pl.ds(i * 64 + u * 16, 16)
            iv = plsc.load_gather(inv_v, [dst_v[sl]])
            a = jnp.exp(e_v[sl] - mvec) * iv
            plsc.addupdate_scatter(w_v, [src_v[sl]], a)
        return 0
    lax.fori_loop(0, VE // 4, p, 0)

    pltpu.sync_copy(w_v, w_hbm.at[wid])


def _sc_heavy_body(src_hbm, dst_hbm, e_hbm, inv_hbm, m_hbm, h_hbm,
                   out_hbm,
                   inv_v, m_v, src_cs, dst_cs, e_cs, rows, acc,
                   gsem0, gsem1, gsem2, ssem0, ssem1, ssem2,
                   isem0, isem1, isem2, isem3, isem4, isem5):
    gsem = [gsem0, gsem1, gsem2]
    ssem = [ssem0, ssem1, ssem2]
    isem = [isem0, isem1, isem2, isem3, isem4, isem5]
    cid = lax.axis_index("c")
    sid = lax.axis_index("s")
    wid = sid * NC + cid
    base = wid * TE
    pltpu.sync_copy(inv_hbm, inv_v)
    pltpu.sync_copy(m_hbm, m_v)
    mvec = m_v[...]

    def zr(k, _):
        for f in range(D // 16):
            rows[0, k, pl.ds(f * 16, 16)] = jnp.zeros((16,), _f32)
        return 0
    lax.fori_loop(0, C, zr, 0)
    for j in range(RPT // RC):
        pltpu.sync_copy(rows.at[0, pl.ds(0, RC)],
                        acc.at[pl.ds(sid * RPT + j * RC, RC)])
    plsc.subcore_barrier()

    def idx_issue(c, qb):
        eb = base + c * C
        pltpu.async_copy(src_hbm.at[pl.ds(eb, C)], src_cs.at[qb], isem[qb])
        pltpu.async_copy(dst_hbm.at[pl.ds(eb, C)], dst_cs.at[qb], isem[qb])
        pltpu.async_copy(e_hbm.at[pl.ds(eb, C)], e_cs.at[qb], isem[qb])

    def idx_wait(c, qb):
        eb = base + c * C
        pltpu.make_async_copy(src_hbm.at[pl.ds(eb, C)], src_cs.at[qb],
                              isem[qb]).wait()
        pltpu.make_async_copy(dst_hbm.at[pl.ds(eb, C)], dst_cs.at[qb],
                              isem[qb]).wait()
        pltpu.make_async_copy(e_hbm.at[pl.ds(eb, C)], e_cs.at[qb],
                              isem[qb]).wait()

    def gather_issue(qb, rb):
        pltpu.async_copy(h_hbm.at[src_cs.at[qb]], rows.at[rb], gsem[rb])

    def gather_wait(qb, rb):
        pltpu.make_async_copy(h_hbm.at[src_cs.at[qb]], rows.at[rb],
                              gsem[rb]).wait()

    def scat_issue(qb, rb):
        pltpu.async_copy(rows.at[rb], acc.at[dst_cs.at[qb]], ssem[rb],
                        add=True)

    def scat_wait(qb, rb):
        pltpu.make_async_copy(rows.at[rb], acc.at[dst_cs.at[qb]],
                              ssem[rb]).wait()

    def scale(qb, rb):
        def g_body(g, _):
            sl16 = pl.ds(g * 16, 16)
            iv = plsc.load_gather(inv_v, [dst_cs[qb, sl16]])
            av16 = jnp.exp(e_cs[qb, sl16] - mvec) * iv
            for k2 in range(16):
                av = jnp.full((16,), av16[k2], _f32)
                def row_mul(gk):
                    for f in range(D // 16):
                        sl = pl.ds(f * 16, 16)
                        rows[rb, gk, sl] = rows[rb, gk, sl] * av
                row_mul(g * 16 + k2)
            return 0
        lax.fori_loop(0, C // 16, g_body, 0)

    def step(c, qb, rb, do_swait, do_gissue, do_iissue, do_iwait):
        if do_iwait:
            idx_wait(c + 1, (qb + 1) % 6)
        if do_swait:
            scat_wait((qb + 4) % 6, (rb + 1) % 3)
        if do_gissue:
            gather_issue((qb + 1) % 6, (rb + 1) % 3)
        if do_iissue:
            idx_issue(c + 3, (qb + 3) % 6)
        gather_wait(qb, rb)
        scale(qb, rb)
        scat_issue(qb, rb)

    idx_issue(0, 0)
    idx_issue(1, 1)
    idx_issue(2, 2)
    idx_wait(0, 0)
    gather_issue(0, 0)

    for c in range(6):
        step(c, c % 6, c % 3, do_swait=(c >= 2), do_gissue=True,
             do_iissue=True, do_iwait=True)

    def body(k, _):
        c0 = k * 6
        for u in range(6):
            step(c0 + u, u, u % 3, do_swait=True, do_gissue=True,
                 do_iissue=True, do_iwait=True)
        return 0
    lax.fori_loop(1, KC // 6 - 1, body, 0)

    for c in range(KC - 6, KC):
        step(c, c % 6, c % 3, do_swait=True,
             do_gissue=(c + 1 < KC),
             do_iissue=(c + 3 < KC),
             do_iwait=(c + 1 < KC))

    scat_wait((KC - 2) % 6, (KC - 2) % 3)
    scat_wait((KC - 1) % 6, (KC - 1) % 3)
    plsc.subcore_barrier()

    for j in range(RPT // RC):
        r = sid * RPT + j * RC
        pltpu.sync_copy(acc.at[pl.ds(r, RC)], rows.at[0, pl.ds(0, RC)])
        pltpu.sync_copy(rows.at[0, pl.ds(0, RC)], out_hbm.at[cid, pl.ds(r, RC)])


_edge_pass = pl.kernel(
    _sc_edge_body,
    out_type=(jax.ShapeDtypeStruct((EP,), _f32),
              jax.ShapeDtypeStruct((NW, NP), _f32),
              jax.ShapeDtypeStruct((NW, 16), _f32)),
    mesh=_sc_mesh,
    compiler_params=_sc_params,
    scratch_types=[pltpu.VMEM((TE,), _i32),
                   pltpu.VMEM((TE,), _i32),
                   pltpu.VMEM((TE,), _f32),
                   pltpu.VMEM((NP,), _f32),
                   pltpu.VMEM((NP,), _f32),
                   pltpu.VMEM((NP,), _f32),
                   pltpu.VMEM((16,), _f32)],
)

_wpass = pl.kernel(
    _sc_w_body,
    out_type=jax.ShapeDtypeStruct((NW, NP), _f32),
    mesh=_sc_mesh,
    compiler_params=_sc_params,
    scratch_types=[pltpu.VMEM((TE,), _i32),
                   pltpu.VMEM((TE,), _i32),
                   pltpu.VMEM((TE,), _f32),
                   pltpu.VMEM((NP,), _f32),
                   pltpu.VMEM((16,), _f32),
                   pltpu.VMEM((NP,), _f32)],
)

_heavy = pl.kernel(
    _sc_heavy_body,
    out_type=jax.ShapeDtypeStruct((NC, NP, D), _f32),
    mesh=_sc_mesh,
    compiler_params=_sc_params,
    scratch_types=[pltpu.VMEM((NP,), _f32),
                   pltpu.VMEM((16,), _f32),
                   pltpu.VMEM((6, C), _i32),
                   pltpu.VMEM((6, C), _i32),
                   pltpu.VMEM((6, C), _f32),
                   pltpu.VMEM((3, C, D), _f32),
                   pltpu.VMEM_SHARED((NP, D), _f32)]
                  + [pltpu.SemaphoreType.DMA] * 12,
)


def kernel(x, edge_index, W1, att_src1, att_dst1, bias1,
           W2, att_src2, att_dst2, bias2):
    ei = edge_index.astype(_i32)
    loop = jnp.arange(N, dtype=_i32)
    pad = jnp.arange(EP - E_TOT, dtype=_i32) % N
    src = jnp.concatenate([ei[0], loop, pad])
    dst = jnp.concatenate([ei[1], loop, pad])
    xp = jnp.pad(x, ((0, NP - N), (0, 0)))

    h1, as1, ad1 = _embed(xp, W1, att_src1.reshape(D, 1), att_dst1.reshape(D, 1))
    e1, dt1, mt1 = _edge_pass(src, dst, as1.reshape(NP), ad1.reshape(NP))
    inv1, m1 = _combine(dt1, mt1)
    outp = _heavy(src, dst, e1, inv1.reshape(NP), m1.reshape(16), h1)

    h2, as2, ad2 = _prep2(outp, bias1.reshape(1, D), W2,
                          att_src2.reshape(D, 1), att_dst2.reshape(D, 1))
    e2, dt2, mt2 = _edge_pass(src, dst, as2.reshape(NP), ad2.reshape(NP))
    inv2, m2 = _combine(dt2, mt2)
    wt = _wpass(src, dst, e2, inv2.reshape(NP), m2.reshape(16))
    out = _final(wt, h2, bias2.reshape(1, D))
    return out.reshape(D)
